# baseline (speedup 1.0000x reference)
import functools

import jax
import jax.numpy as jnp
from jax.experimental import pallas as pl
from jax.experimental.pallas import tpu as pltpu

SMALL = -1000.0
B, T, D, L = 128, 512, 256, 16
T_BLK = 64
N_BLK = T // T_BLK
NORM_EVERY = 16


def _sub_reduce(x):
    s = x
    for sh in (8, 4, 2, 1):
        s = s + pltpu.roll(s, sh, axis=0)
    return s[0:1, :]


def _crf_body(x_ref, tags_ref, slt_ref, wt_ref, b_ref, trans_ref,
              out_ref,
              alpha_ref, scale_ref, logz_ref, real_ref, carry_ref):
    g = pl.program_id(0)
    t_base = g * T_BLK

    seq_len_t = slt_ref[...]
    trans = trans_ref[...]
    exp_t = jnp.exp(trans)
    sub_lb = jax.lax.broadcasted_iota(jnp.int32, (L, B), 0)

    @pl.when(g == 0)
    def _init():
        start_oh = (sub_lb == (L - 2)).astype(jnp.float32)
        alpha_ref[...] = start_oh
        carry_ref[...] = start_oh
        scale_ref[...] = jnp.zeros((1, B), jnp.float32)
        logz_ref[...] = jnp.zeros((1, B), jnp.float32)
        real_ref[...] = jnp.zeros((1, B), jnp.float32)

    x2d = x_ref[...].reshape(B * T_BLK, D)
    pred2d = jnp.dot(x2d, wt_ref[...], preferred_element_type=jnp.float32)
    pred2d = pred2d + b_ref[...]
    pred3 = pred2d.reshape(B, T_BLK, L)

    eye = (jax.lax.broadcasted_iota(jnp.int32, (L, L), 0)
           == jax.lax.broadcasted_iota(jnp.int32, (L, L), 1)
           ).astype(jnp.float32)
    rot_c = []
    for k in range(L):
        rk = pltpu.roll(exp_t, L - k, axis=0) if k else exp_t
        diag = jnp.sum(rk * eye, axis=1, keepdims=True)
        rot_c.append(jnp.broadcast_to(diag, (L, B)))

    tags_f = tags_ref[...]
    sub_f = sub_lb.astype(jnp.float32)
    trans_tt = jnp.transpose(trans)

    alpha = alpha_ref[...]
    scale = scale_ref[...]
    acc2 = jnp.zeros((L, B), jnp.float32)
    teacc = jnp.zeros((1, B), jnp.float32)
    ohp = carry_ref[...]
    cap_rows = []
    nrms = []
    for i in range(T_BLK):
        t = t_base + i
        p_t = jnp.transpose(pred3[:, i, :])
        ohc = (sub_f == tags_f[i:i + 1, :]).astype(jnp.float32)
        m = jnp.dot(trans_tt, ohp, preferred_element_type=jnp.float32)
        in_seq = (seq_len_t > t).astype(jnp.float32)
        was_last = (seq_len_t == t).astype(jnp.float32)
        acc2 = acc2 + in_seq * ((p_t + m) * ohc)
        teacc = teacc + was_last * m[L - 1:L, :]
        ohp = ohc
        terms = [(pltpu.roll(alpha, L - k, axis=0) if k else alpha) * rot_c[k]
                 for k in range(L)]
        while len(terms) > 1:
            terms = [terms[j] + terms[j + 1] for j in range(0, len(terms), 2)]
        a1 = terms[0]
        cap_rows.append(a1[L - 1:L, :])
        alpha = a1 * jnp.exp(p_t)
        if i % NORM_EVERY == NORM_EVERY - 1:
            nrm = jnp.max(alpha, axis=0, keepdims=True)
            alpha = alpha * (1.0 / nrm)
            nrms.append(nrm)
    carry_ref[...] = ohp
    real_ref[...] = real_ref[...] + _sub_reduce(acc2) + teacc

    t_idx = t_base + jax.lax.broadcasted_iota(jnp.int32, (T_BLK, B), 0)
    caps_raw = jnp.concatenate(cap_rows, axis=0)
    sub32 = jax.lax.broadcasted_iota(jnp.int32, (T_BLK, B), 0)
    lognrm = jnp.log(jnp.concatenate(nrms, axis=0))
    grpadj = jnp.zeros((T_BLK, B), jnp.float32)
    for gi in range(len(nrms) - 1):
        boundary = (gi + 1) * NORM_EVERY
        grpadj = grpadj + jnp.where(sub32 >= boundary,
                                    lognrm[gi:gi + 1, :], 0.0)
    caps = scale + grpadj + jnp.log(caps_raw)
    logz_ref[...] = logz_ref[...] + jnp.sum(
        jnp.where(t_idx == seq_len_t, caps, 0.0), axis=0, keepdims=True)

    alpha_ref[...] = alpha
    scale_ref[...] = scale + jnp.sum(lognrm, axis=0, keepdims=True)

    @pl.when(g == N_BLK - 1)
    def _fin():
        out_ref[...] = jnp.sum(logz_ref[...] - real_ref[...],
                               keepdims=True)


@functools.partial(jax.jit, static_argnames=())
def kernel(input, tags, seq_len, W, b, transitions):
    tags_t = tags.T.astype(jnp.float32)
    seqlen_t = seq_len.reshape(1, B).astype(jnp.int32)
    lane_l = jnp.arange(L)
    wt = jnp.where(lane_l[None, :] >= L - 2, 0.0,
                   W.T).astype(jnp.bfloat16)
    b2 = jnp.where(lane_l >= L - 2, SMALL,
                   b.astype(jnp.float32)).reshape(1, L)
    x_bf = input.astype(jnp.bfloat16)

    out = pl.pallas_call(
        _crf_body,
        grid=(N_BLK,),
        in_specs=[
            pl.BlockSpec((B, T_BLK, D), lambda g: (0, g, 0)),
            pl.BlockSpec((T_BLK, B), lambda g: (g, 0)),
            pl.BlockSpec((1, B), lambda g: (0, 0)),
            pl.BlockSpec((D, L), lambda g: (0, 0)),
            pl.BlockSpec((1, L), lambda g: (0, 0)),
            pl.BlockSpec((L, L), lambda g: (0, 0)),
        ],
        out_specs=pl.BlockSpec((1, 1), lambda g: (0, 0)),
        out_shape=jax.ShapeDtypeStruct((1, 1), jnp.float32),
        scratch_shapes=[
            pltpu.VMEM((L, B), jnp.float32),
            pltpu.VMEM((1, B), jnp.float32),
            pltpu.VMEM((1, B), jnp.float32),
            pltpu.VMEM((1, B), jnp.float32),
            pltpu.VMEM((L, B), jnp.float32),
        ],
    )(x_bf, tags_t, seqlen_t, wt, b2, transitions)
    return out[0, 0]


# speedup vs baseline: 1.6724x; 1.6724x over previous
import functools

import jax
import jax.numpy as jnp
from jax.experimental import pallas as pl
from jax.experimental.pallas import tpu as pltpu

SMALL = -1000.0
B, T, D, L = 128, 512, 256, 16
T_BLK = 32
N_BLK = T // T_BLK
NORM_EVERY = 16


def _sub_reduce(x):
    s = x
    for sh in (8, 4, 2, 1):
        s = s + pltpu.roll(s, sh, axis=0)
    return s[0:1, :]


def _crf_body(x_ref, tags_ref, slt_ref, wt_ref, b_ref, trans_ref,
              out_ref,
              alpha_ref, scale_ref, logz_ref, real_ref, carry_ref):
    g = pl.program_id(0)
    t_base = g * T_BLK

    seq_len_t = slt_ref[...]
    trans = trans_ref[...]
    exp_t = jnp.exp(trans)
    sub_lb = jax.lax.broadcasted_iota(jnp.int32, (L, B), 0)

    @pl.when(g == 0)
    def _init():
        start_oh = (sub_lb == (L - 2)).astype(jnp.float32)
        alpha_ref[...] = start_oh
        carry_ref[...] = start_oh
        scale_ref[...] = jnp.zeros((1, B), jnp.float32)
        logz_ref[...] = jnp.zeros((1, B), jnp.float32)
        real_ref[...] = jnp.zeros((1, B), jnp.float32)

    x2d = x_ref[...].reshape(B * T_BLK, D).astype(jnp.bfloat16)
    pred2d = jnp.dot(x2d, wt_ref[...], preferred_element_type=jnp.float32)
    pred2d = pred2d + b_ref[...]
    pred3 = pred2d.reshape(B, T_BLK, L)

    eye = (jax.lax.broadcasted_iota(jnp.int32, (L, L), 0)
           == jax.lax.broadcasted_iota(jnp.int32, (L, L), 1)
           ).astype(jnp.float32)
    rot_c = []
    for k in range(L):
        rk = pltpu.roll(exp_t, L - k, axis=0) if k else exp_t
        diag = jnp.sum(rk * eye, axis=1, keepdims=True)
        rot_c.append(jnp.broadcast_to(diag, (L, B)))

    tags_f = tags_ref[...]
    sub_f = sub_lb.astype(jnp.float32)
    trans_tt = jnp.transpose(trans)

    alpha = alpha_ref[...]
    scale = scale_ref[...]
    acc2 = jnp.zeros((L, B), jnp.float32)
    teacc = jnp.zeros((1, B), jnp.float32)
    ohp = carry_ref[...]
    cap_rows = []
    nrms = []
    for i in range(T_BLK):
        t = t_base + i
        p_t = jnp.transpose(pred3[:, i, :])
        ohc = (sub_f == tags_f[i:i + 1, :]).astype(jnp.float32)
        m = jnp.dot(trans_tt, ohp, preferred_element_type=jnp.float32)
        in_seq = (seq_len_t > t).astype(jnp.float32)
        was_last = (seq_len_t == t).astype(jnp.float32)
        acc2 = acc2 + in_seq * ((p_t + m) * ohc)
        teacc = teacc + was_last * m[L - 1:L, :]
        ohp = ohc
        terms = [(pltpu.roll(alpha, L - k, axis=0) if k else alpha) * rot_c[k]
                 for k in range(L)]
        while len(terms) > 1:
            terms = [terms[j] + terms[j + 1] for j in range(0, len(terms), 2)]
        a1 = terms[0]
        cap_rows.append(a1[L - 1:L, :])
        alpha = a1 * jnp.exp(p_t)
        if i % NORM_EVERY == NORM_EVERY - 1:
            nrm = jnp.max(alpha, axis=0, keepdims=True)
            alpha = alpha * (1.0 / nrm)
            nrms.append(nrm)
    carry_ref[...] = ohp
    real_ref[...] = real_ref[...] + _sub_reduce(acc2) + teacc

    t_idx = t_base + jax.lax.broadcasted_iota(jnp.int32, (T_BLK, B), 0)
    caps_raw = jnp.concatenate(cap_rows, axis=0)
    sub32 = jax.lax.broadcasted_iota(jnp.int32, (T_BLK, B), 0)
    lognrm = jnp.log(jnp.concatenate(nrms, axis=0))
    grpadj = jnp.zeros((T_BLK, B), jnp.float32)
    for gi in range(len(nrms) - 1):
        boundary = (gi + 1) * NORM_EVERY
        grpadj = grpadj + jnp.where(sub32 >= boundary,
                                    lognrm[gi:gi + 1, :], 0.0)
    caps = scale + grpadj + jnp.log(caps_raw)
    logz_ref[...] = logz_ref[...] + jnp.sum(
        jnp.where(t_idx == seq_len_t, caps, 0.0), axis=0, keepdims=True)

    alpha_ref[...] = alpha
    scale_ref[...] = scale + jnp.sum(lognrm, axis=0, keepdims=True)

    @pl.when(g == N_BLK - 1)
    def _fin():
        out_ref[...] = jnp.sum(logz_ref[...] - real_ref[...],
                               keepdims=True)


@functools.partial(jax.jit, static_argnames=())
def kernel(input, tags, seq_len, W, b, transitions):
    tags_t = tags.T.astype(jnp.float32)
    seqlen_t = seq_len.reshape(1, B).astype(jnp.int32)
    lane_l = jnp.arange(L)
    wt = jnp.where(lane_l[None, :] >= L - 2, 0.0,
                   W.T).astype(jnp.bfloat16)
    b2 = jnp.where(lane_l >= L - 2, SMALL,
                   b.astype(jnp.float32)).reshape(1, L)

    out = pl.pallas_call(
        _crf_body,
        grid=(N_BLK,),
        in_specs=[
            pl.BlockSpec((B, T_BLK, D), lambda g: (0, g, 0)),
            pl.BlockSpec((T_BLK, B), lambda g: (g, 0)),
            pl.BlockSpec((1, B), lambda g: (0, 0)),
            pl.BlockSpec((D, L), lambda g: (0, 0)),
            pl.BlockSpec((1, L), lambda g: (0, 0)),
            pl.BlockSpec((L, L), lambda g: (0, 0)),
        ],
        out_specs=pl.BlockSpec((1, 1), lambda g: (0, 0)),
        out_shape=jax.ShapeDtypeStruct((1, 1), jnp.float32),
        scratch_shapes=[
            pltpu.VMEM((L, B), jnp.float32),
            pltpu.VMEM((1, B), jnp.float32),
            pltpu.VMEM((1, B), jnp.float32),
            pltpu.VMEM((1, B), jnp.float32),
            pltpu.VMEM((L, B), jnp.float32),
        ],
    )(input, tags_t, seqlen_t, wt, b2, transitions)
    return out[0, 0]


# speedup vs baseline: 1.7363x; 1.0382x over previous
import functools

import jax
import jax.numpy as jnp
from jax.experimental import pallas as pl
from jax.experimental.pallas import tpu as pltpu

SMALL = -1000.0
B, T, D, L = 128, 512, 256, 16
T_BLK = 32
N_BLK = T // T_BLK
NORM_EVERY = 16


def _sub_reduce(x):
    s = x
    for sh in (8, 4, 2, 1):
        s = s + pltpu.roll(s, sh, axis=0)
    return s[0:1, :]


def _crf_body(x_ref, tags_ref, slt_ref, wt_ref, b_ref, trans_ref,
              out_ref,
              alpha_ref, scale_ref, logz_ref, real_ref, carry_ref):
    g = pl.program_id(0)
    t_base = g * T_BLK

    seq_len_t = slt_ref[...]
    trans = trans_ref[...]
    exp_t = jnp.exp(trans)
    sub_lb = jax.lax.broadcasted_iota(jnp.int32, (L, B), 0)

    @pl.when(g == 0)
    def _init():
        start_oh = (sub_lb == (L - 2)).astype(jnp.float32)
        alpha_ref[...] = start_oh
        carry_ref[...] = start_oh
        scale_ref[...] = jnp.zeros((1, B), jnp.float32)
        logz_ref[...] = jnp.zeros((1, B), jnp.float32)
        real_ref[...] = jnp.zeros((1, B), jnp.float32)

    x2d = x_ref[...].reshape(B * T_BLK, D)
    pred2d = jnp.dot(x2d, wt_ref[...], preferred_element_type=jnp.float32)
    pred2d = pred2d + b_ref[...]
    pred3 = pred2d.reshape(B, T_BLK, L)

    eye = (jax.lax.broadcasted_iota(jnp.int32, (L, L), 0)
           == jax.lax.broadcasted_iota(jnp.int32, (L, L), 1)
           ).astype(jnp.float32)
    rot_c = []
    for k in range(L):
        rk = pltpu.roll(exp_t, L - k, axis=0) if k else exp_t
        diag = jnp.sum(rk * eye, axis=1, keepdims=True)
        rot_c.append(jnp.broadcast_to(diag, (L, B)))

    tags_f = tags_ref[...]
    sub_f = sub_lb.astype(jnp.float32)
    trans_tt = jnp.transpose(trans)

    alpha = alpha_ref[...]
    scale = scale_ref[...]
    acc2 = jnp.zeros((L, B), jnp.float32)
    teacc = jnp.zeros((1, B), jnp.float32)
    ohp = carry_ref[...]
    cap_rows = []
    nrms = []
    for i in range(T_BLK):
        t = t_base + i
        p_t = jnp.transpose(pred3[:, i, :])
        ohc = (sub_f == tags_f[i:i + 1, :]).astype(jnp.float32)
        m = jnp.dot(trans_tt, ohp, preferred_element_type=jnp.float32)
        in_seq = (seq_len_t > t).astype(jnp.float32)
        was_last = (seq_len_t == t).astype(jnp.float32)
        acc2 = acc2 + in_seq * ((p_t + m) * ohc)
        teacc = teacc + was_last * m[L - 1:L, :]
        ohp = ohc
        terms = [(pltpu.roll(alpha, L - k, axis=0) if k else alpha) * rot_c[k]
                 for k in range(L)]
        while len(terms) > 1:
            terms = [terms[j] + terms[j + 1] for j in range(0, len(terms), 2)]
        a1 = terms[0]
        cap_rows.append(a1[L - 1:L, :])
        alpha = a1 * jnp.exp(p_t)
        if i % NORM_EVERY == NORM_EVERY - 1:
            nrm = jnp.max(alpha, axis=0, keepdims=True)
            alpha = alpha * (1.0 / nrm)
            nrms.append(nrm)
    carry_ref[...] = ohp
    real_ref[...] = real_ref[...] + _sub_reduce(acc2) + teacc

    t_idx = t_base + jax.lax.broadcasted_iota(jnp.int32, (T_BLK, B), 0)
    caps_raw = jnp.concatenate(cap_rows, axis=0)
    sub32 = jax.lax.broadcasted_iota(jnp.int32, (T_BLK, B), 0)
    lognrm = jnp.log(jnp.concatenate(nrms, axis=0))
    grpadj = jnp.zeros((T_BLK, B), jnp.float32)
    for gi in range(len(nrms) - 1):
        boundary = (gi + 1) * NORM_EVERY
        grpadj = grpadj + jnp.where(sub32 >= boundary,
                                    lognrm[gi:gi + 1, :], 0.0)
    caps = scale + grpadj + jnp.log(caps_raw)
    logz_ref[...] = logz_ref[...] + jnp.sum(
        jnp.where(t_idx == seq_len_t, caps, 0.0), axis=0, keepdims=True)

    alpha_ref[...] = alpha
    scale_ref[...] = scale + jnp.sum(lognrm, axis=0, keepdims=True)

    @pl.when(g == N_BLK - 1)
    def _fin():
        out_ref[...] = jnp.sum(logz_ref[...] - real_ref[...],
                               keepdims=True)


@functools.partial(jax.jit, static_argnames=())
def kernel(input, tags, seq_len, W, b, transitions):
    tags_t = tags.T.astype(jnp.float32)
    seqlen_t = seq_len.reshape(1, B).astype(jnp.int32)
    lane_l = jnp.arange(L)
    wt = jnp.where(lane_l[None, :] >= L - 2, 0.0,
                   W.astype(jnp.float32).T)
    b2 = jnp.where(lane_l >= L - 2, SMALL,
                   b.astype(jnp.float32)).reshape(1, L)

    out = pl.pallas_call(
        _crf_body,
        grid=(N_BLK,),
        in_specs=[
            pl.BlockSpec((B, T_BLK, D), lambda g: (0, g, 0)),
            pl.BlockSpec((T_BLK, B), lambda g: (g, 0)),
            pl.BlockSpec((1, B), lambda g: (0, 0)),
            pl.BlockSpec((D, L), lambda g: (0, 0)),
            pl.BlockSpec((1, L), lambda g: (0, 0)),
            pl.BlockSpec((L, L), lambda g: (0, 0)),
        ],
        out_specs=pl.BlockSpec((1, 1), lambda g: (0, 0)),
        out_shape=jax.ShapeDtypeStruct((1, 1), jnp.float32),
        scratch_shapes=[
            pltpu.VMEM((L, B), jnp.float32),
            pltpu.VMEM((1, B), jnp.float32),
            pltpu.VMEM((1, B), jnp.float32),
            pltpu.VMEM((1, B), jnp.float32),
            pltpu.VMEM((L, B), jnp.float32),
        ],
    )(input, tags_t, seqlen_t, wt, b2, transitions)
    return out[0, 0]


# speedup vs baseline: 1.8111x; 1.0430x over previous
import functools

import jax
import jax.numpy as jnp
from jax.experimental import pallas as pl
from jax.experimental.pallas import tpu as pltpu

SMALL = -1000.0
B, T, D, L = 128, 512, 256, 16
T_BLK = 64
N_BLK = T // T_BLK
NORM_EVERY = 16


def _sub_reduce(x):
    s = x
    for sh in (8, 4, 2, 1):
        s = s + pltpu.roll(s, sh, axis=0)
    return s[0:1, :]


def _crf_body(x_ref, tags_ref, slt_ref, wt_ref, b_ref, trans_ref,
              out_ref,
              alpha_ref, scale_ref, logz_ref, real_ref, carry_ref):
    g = pl.program_id(0)
    t_base = g * T_BLK

    seq_len_t = slt_ref[...]
    trans = trans_ref[...]
    exp_t = jnp.exp(trans)
    sub_lb = jax.lax.broadcasted_iota(jnp.int32, (L, B), 0)

    @pl.when(g == 0)
    def _init():
        start_oh = (sub_lb == (L - 2)).astype(jnp.float32)
        alpha_ref[...] = start_oh
        carry_ref[...] = start_oh
        scale_ref[...] = jnp.zeros((1, B), jnp.float32)
        logz_ref[...] = jnp.zeros((1, B), jnp.float32)
        real_ref[...] = jnp.zeros((1, B), jnp.float32)

    x2d = x_ref[...].reshape(B * T_BLK, D)
    pred2d = jnp.dot(x2d, wt_ref[...], preferred_element_type=jnp.float32)
    pred2d = pred2d + b_ref[...]
    pred3 = pred2d.reshape(B, T_BLK, L)

    eye = (jax.lax.broadcasted_iota(jnp.int32, (L, L), 0)
           == jax.lax.broadcasted_iota(jnp.int32, (L, L), 1)
           ).astype(jnp.float32)
    rot_c = []
    for k in range(L):
        rk = pltpu.roll(exp_t, L - k, axis=0) if k else exp_t
        diag = jnp.sum(rk * eye, axis=1, keepdims=True)
        rot_c.append(jnp.broadcast_to(diag, (L, B)))

    tags_f = tags_ref[...]
    sub_f = sub_lb.astype(jnp.float32)
    trans_tt = jnp.transpose(trans)

    alpha = alpha_ref[...]
    scale = scale_ref[...]
    acc2 = jnp.zeros((L, B), jnp.float32)
    teacc = jnp.zeros((1, B), jnp.float32)
    ohp = carry_ref[...]
    cap_rows = []
    nrms = []
    for i in range(T_BLK):
        t = t_base + i
        p_t = jnp.transpose(pred3[:, i, :])
        ohc = (sub_f == tags_f[i:i + 1, :]).astype(jnp.float32)
        m = jnp.dot(trans_tt, ohp, preferred_element_type=jnp.float32)
        in_seq = (seq_len_t > t).astype(jnp.float32)
        was_last = (seq_len_t == t).astype(jnp.float32)
        acc2 = acc2 + in_seq * ((p_t + m) * ohc)
        teacc = teacc + was_last * m[L - 1:L, :]
        ohp = ohc
        terms = [(pltpu.roll(alpha, L - k, axis=0) if k else alpha) * rot_c[k]
                 for k in range(L)]
        while len(terms) > 1:
            terms = [terms[j] + terms[j + 1] for j in range(0, len(terms), 2)]
        a1 = terms[0]
        cap_rows.append(a1[L - 1:L, :])
        alpha = a1 * jnp.exp(p_t)
        if i % NORM_EVERY == NORM_EVERY - 1:
            nrm = jnp.max(alpha, axis=0, keepdims=True)
            alpha = alpha * (1.0 / nrm)
            nrms.append(nrm)
    carry_ref[...] = ohp
    real_ref[...] = real_ref[...] + _sub_reduce(acc2) + teacc

    t_idx = t_base + jax.lax.broadcasted_iota(jnp.int32, (T_BLK, B), 0)
    caps_raw = jnp.concatenate(cap_rows, axis=0)
    sub32 = jax.lax.broadcasted_iota(jnp.int32, (T_BLK, B), 0)
    lognrm = jnp.log(jnp.concatenate(nrms, axis=0))
    grpadj = jnp.zeros((T_BLK, B), jnp.float32)
    for gi in range(len(nrms) - 1):
        boundary = (gi + 1) * NORM_EVERY
        grpadj = grpadj + jnp.where(sub32 >= boundary,
                                    lognrm[gi:gi + 1, :], 0.0)
    caps = scale + grpadj + jnp.log(caps_raw)
    logz_ref[...] = logz_ref[...] + jnp.sum(
        jnp.where(t_idx == seq_len_t, caps, 0.0), axis=0, keepdims=True)

    alpha_ref[...] = alpha
    scale_ref[...] = scale + jnp.sum(lognrm, axis=0, keepdims=True)

    @pl.when(g == N_BLK - 1)
    def _fin():
        out_ref[...] = jnp.sum(logz_ref[...] - real_ref[...],
                               keepdims=True)


@functools.partial(jax.jit, static_argnames=())
def kernel(input, tags, seq_len, W, b, transitions):
    tags_t = tags.T.astype(jnp.float32)
    seqlen_t = seq_len.reshape(1, B).astype(jnp.int32)
    lane_l = jnp.arange(L)
    wt = jnp.where(lane_l[None, :] >= L - 2, 0.0,
                   W.astype(jnp.float32).T)
    b2 = jnp.where(lane_l >= L - 2, SMALL,
                   b.astype(jnp.float32)).reshape(1, L)

    out = pl.pallas_call(
        _crf_body,
        grid=(N_BLK,),
        in_specs=[
            pl.BlockSpec((B, T_BLK, D), lambda g: (0, g, 0)),
            pl.BlockSpec((T_BLK, B), lambda g: (g, 0)),
            pl.BlockSpec((1, B), lambda g: (0, 0)),
            pl.BlockSpec((D, L), lambda g: (0, 0)),
            pl.BlockSpec((1, L), lambda g: (0, 0)),
            pl.BlockSpec((L, L), lambda g: (0, 0)),
        ],
        out_specs=pl.BlockSpec((1, 1), lambda g: (0, 0)),
        out_shape=jax.ShapeDtypeStruct((1, 1), jnp.float32),
        scratch_shapes=[
            pltpu.VMEM((L, B), jnp.float32),
            pltpu.VMEM((1, B), jnp.float32),
            pltpu.VMEM((1, B), jnp.float32),
            pltpu.VMEM((1, B), jnp.float32),
            pltpu.VMEM((L, B), jnp.float32),
        ],
    )(input, tags_t, seqlen_t, wt, b2, transitions)
    return out[0, 0]
